# baseline (speedup 1.0000x reference)
import jax
import jax.numpy as jnp
from jax import lax
from jax.experimental import pallas as pl
from jax.experimental.pallas import tpu as pltpu
from jax.experimental.pallas import tpu_sc as plsc

VOCAB = 1000000
EMB_DIM = 64
BATCH = 1024
SEQ = 200

NUM_IDX = BATCH * SEQ
WINDOW = 128
NUM_CHUNKS = NUM_IDX // WINDOW
NW = 32
CHUNKS_PER_W = NUM_CHUNKS // NW
NBUF = 5
PAIR_DIM = 2 * EMB_DIM


def _gather_fn():
    mesh = plsc.VectorSubcoreMesh(
        core_axis_name="core", subcore_axis_name="subcore"
    )

    @jax.jit
    def gather(table2, idxp):
        @pl.kernel(
            out_type=jax.ShapeDtypeStruct((NUM_IDX, PAIR_DIM), table2.dtype),
            mesh=mesh,
            scratch_types=[
                pltpu.VMEM((CHUNKS_PER_W * WINDOW,), jnp.int32),
                pltpu.VMEM((NBUF, WINDOW, PAIR_DIM), jnp.float32),
                pltpu.SemaphoreType.DMA,
                pltpu.SemaphoreType.DMA((NBUF,)),
                pltpu.SemaphoreType.DMA((NBUF,)),
            ],
            compiler_params=pltpu.CompilerParams(use_tc_tiling_on_sc=False),
        )
        def kernel(x_hbm, i_hbm, o_hbm, idx_v, rows_v, isem, gsem, ssem):
            cid = lax.axis_index("core")
            sid = lax.axis_index("subcore")
            wid = sid * 2 + cid
            base = wid * CHUNKS_PER_W * WINDOW

            pltpu.async_copy(
                i_hbm.at[pl.ds(base, CHUNKS_PER_W * WINDOW)], idx_v, isem
            ).wait()

            for b in range(NBUF):
                pltpu.async_copy(
                    x_hbm.at[idx_v.at[pl.ds(b * WINDOW, WINDOW)]],
                    rows_v.at[b],
                    gsem.at[b],
                )

            @pl.loop(0, CHUNKS_PER_W, step=NBUF)
            def _(j0):
                for b in range(NBUF):
                    j = j0 + b
                    pltpu.make_async_copy(
                        x_hbm.at[idx_v.at[pl.ds(b * WINDOW, WINDOW)]],
                        rows_v.at[b],
                        gsem.at[b],
                    ).wait()
                    out_slice = o_hbm.at[pl.ds(base + j * WINDOW, WINDOW)]
                    pltpu.async_copy(rows_v.at[b], out_slice, ssem.at[b])
                    pltpu.make_async_copy(
                        rows_v.at[b], out_slice, ssem.at[b]
                    ).wait()

                    @pl.when(j + NBUF < CHUNKS_PER_W)
                    def _():
                        pltpu.async_copy(
                            x_hbm.at[
                                idx_v.at[pl.ds((j + NBUF) * WINDOW, WINDOW)]
                            ],
                            rows_v.at[b],
                            gsem.at[b],
                        )

        return kernel(table2, idxp)

    return gather


_gather = _gather_fn()


def kernel(inp, emb_weight):
    table2 = emb_weight.reshape(VOCAB // 2, PAIR_DIM)
    flat_idx = inp.reshape(-1)
    pairs = _gather(table2, flat_idx >> 1)
    odd = (flat_idx & 1).astype(jnp.bool_)
    out = jnp.where(odd[:, None], pairs[:, EMB_DIM:], pairs[:, :EMB_DIM])
    return out.reshape(BATCH, SEQ, EMB_DIM)


# speedup vs baseline: 1.1205x; 1.1205x over previous
import jax
import jax.numpy as jnp
from jax import lax
from jax.experimental import pallas as pl
from jax.experimental.pallas import tpu as pltpu
from jax.experimental.pallas import tpu_sc as plsc

VOCAB = 1000000
EMB_DIM = 64
BATCH = 1024
SEQ = 200

NW = 32
ROWS_PER_W = BATCH // NW
W0 = 128
W1 = SEQ - W0
NBUF = 4


def _gather_fn():
    mesh = plsc.VectorSubcoreMesh(
        core_axis_name="core", subcore_axis_name="subcore"
    )

    @jax.jit
    def gather(table, idx):
        @pl.kernel(
            out_type=jax.ShapeDtypeStruct((BATCH, SEQ, EMB_DIM), table.dtype),
            mesh=mesh,
            scratch_types=[
                pltpu.VMEM((ROWS_PER_W, SEQ), jnp.int32),
                pltpu.VMEM((NBUF, SEQ, EMB_DIM), jnp.float32),
                pltpu.SemaphoreType.DMA,
                pltpu.SemaphoreType.DMA((NBUF,)),
                pltpu.SemaphoreType.DMA((NBUF,)),
            ],
            compiler_params=pltpu.CompilerParams(use_tc_tiling_on_sc=False),
        )
        def kernel(x_hbm, i_hbm, o_hbm, idx_v, rows_v, isem, gsem, ssem):
            cid = lax.axis_index("core")
            sid = lax.axis_index("subcore")
            wid = sid * 2 + cid
            row0 = wid * ROWS_PER_W

            pltpu.async_copy(
                i_hbm.at[pl.ds(row0, ROWS_PER_W)], idx_v, isem
            ).wait()

            def start_row(r, b):
                pltpu.async_copy(
                    x_hbm.at[idx_v.at[r, pl.ds(0, W0)]],
                    rows_v.at[b, pl.ds(0, W0)],
                    gsem.at[b],
                )
                pltpu.async_copy(
                    x_hbm.at[idx_v.at[r, pl.ds(W0, W1)]],
                    rows_v.at[b, pl.ds(W0, W1)],
                    gsem.at[b],
                )

            def wait_row(r, b):
                pltpu.make_async_copy(
                    x_hbm.at[idx_v.at[r, pl.ds(0, W0)]],
                    rows_v.at[b, pl.ds(0, W0)],
                    gsem.at[b],
                ).wait()
                pltpu.make_async_copy(
                    x_hbm.at[idx_v.at[r, pl.ds(W0, W1)]],
                    rows_v.at[b, pl.ds(W0, W1)],
                    gsem.at[b],
                ).wait()

            for b in range(NBUF):
                start_row(b, b)

            @pl.loop(0, ROWS_PER_W, step=NBUF)
            def _(r0):
                for b in range(NBUF):
                    r = r0 + b
                    wait_row(r, b)
                    out_slice = o_hbm.at[row0 + r]
                    pltpu.async_copy(rows_v.at[b], out_slice, ssem.at[b])
                    pltpu.make_async_copy(
                        rows_v.at[b], out_slice, ssem.at[b]
                    ).wait()

                    @pl.when(r + NBUF < ROWS_PER_W)
                    def _():
                        start_row(r + NBUF, b)

        return kernel(table, idx)

    return gather


_gather = _gather_fn()


def kernel(inp, emb_weight):
    return _gather(emb_weight, inp)


# speedup vs baseline: 1.1244x; 1.0035x over previous
import jax
import jax.numpy as jnp
from jax import lax
from jax.experimental import pallas as pl
from jax.experimental.pallas import tpu as pltpu
from jax.experimental.pallas import tpu_sc as plsc

VOCAB = 1000000
EMB_DIM = 64
BATCH = 1024
SEQ = 200

NW = 32
ROWS_PER_W = BATCH // NW
W0 = 128
W1 = SEQ - W0
NBUF = 4


def _gather_fn():
    mesh = plsc.VectorSubcoreMesh(
        core_axis_name="core", subcore_axis_name="subcore"
    )

    @jax.jit
    def gather(table, idx):
        @pl.kernel(
            out_type=jax.ShapeDtypeStruct((BATCH, SEQ, EMB_DIM), table.dtype),
            mesh=mesh,
            scratch_types=[
                pltpu.VMEM((ROWS_PER_W, SEQ), jnp.int32),
                pltpu.VMEM((NBUF, SEQ, EMB_DIM), jnp.float32),
                pltpu.SemaphoreType.DMA,
                pltpu.SemaphoreType.DMA((NBUF,)),
                pltpu.SemaphoreType.DMA((NBUF,)),
            ],
            compiler_params=pltpu.CompilerParams(use_tc_tiling_on_sc=False),
        )
        def kernel(x_hbm, i_hbm, o_hbm, idx_v, rows_v, isem, gsem, ssem):
            cid = lax.axis_index("core")
            sid = lax.axis_index("subcore")
            wid = sid * 2 + cid
            row0 = wid * ROWS_PER_W

            pltpu.async_copy(
                i_hbm.at[pl.ds(row0, ROWS_PER_W)], idx_v, isem
            ).wait()

            def start_row(r, b):
                pltpu.async_copy(
                    x_hbm.at[idx_v.at[r, pl.ds(0, W0)]],
                    rows_v.at[b, pl.ds(0, W0)],
                    gsem.at[b],
                )
                pltpu.async_copy(
                    x_hbm.at[idx_v.at[r, pl.ds(W0, W1)]],
                    rows_v.at[b, pl.ds(W0, W1)],
                    gsem.at[b],
                )

            def wait_row(r, b):
                pltpu.make_async_copy(
                    x_hbm.at[idx_v.at[r, pl.ds(0, W0)]],
                    rows_v.at[b, pl.ds(0, W0)],
                    gsem.at[b],
                ).wait()
                pltpu.make_async_copy(
                    x_hbm.at[idx_v.at[r, pl.ds(W0, W1)]],
                    rows_v.at[b, pl.ds(W0, W1)],
                    gsem.at[b],
                ).wait()

            for b in range(NBUF):
                start_row(b, b)

            @pl.loop(0, ROWS_PER_W, step=NBUF)
            def _(r0):
                for b in range(NBUF):
                    r = r0 + b
                    wait_row(r, b)
                    out_slice = o_hbm.at[row0 + r]
                    pltpu.async_copy(rows_v.at[b], out_slice, ssem.at[b])
                    pltpu.make_async_copy(
                        rows_v.at[b], out_slice, ssem.at[b]
                    ).wait()

                    @pl.when(r + NBUF < ROWS_PER_W)
                    def _():
                        start_row(r + NBUF, b)

        return kernel(table, idx)

    return gather


_gather = _gather_fn()


def kernel(inp, emb_weight):
    idx = jnp.clip(inp, 0, VOCAB - 1)
    return _gather(emb_weight, idx)


# speedup vs baseline: 1.1865x; 1.0553x over previous
import jax
import jax.numpy as jnp
from jax import lax
from jax.experimental import pallas as pl
from jax.experimental.pallas import tpu as pltpu
from jax.experimental.pallas import tpu_sc as plsc

VOCAB = 1000000
EMB_DIM = 64
BATCH = 1024
SEQ = 200

NW = 32
ROWS_PER_W = BATCH // NW
W0 = 128
W1 = SEQ - W0
NBUF = 4


def _gather_fn():
    mesh = plsc.VectorSubcoreMesh(
        core_axis_name="core", subcore_axis_name="subcore"
    )

    @jax.jit
    def gather(table, idx):
        @pl.kernel(
            out_type=jax.ShapeDtypeStruct((BATCH, SEQ, EMB_DIM), table.dtype),
            mesh=mesh,
            scratch_types=[
                pltpu.VMEM((ROWS_PER_W, SEQ), jnp.int32),
                pltpu.VMEM((NBUF, SEQ, 2 * EMB_DIM), jnp.float32),
                pltpu.SemaphoreType.DMA,
                pltpu.SemaphoreType.DMA((NBUF,)),
                pltpu.SemaphoreType.DMA((NBUF,)),
            ],
            compiler_params=pltpu.CompilerParams(use_tc_tiling_on_sc=False),
        )
        def kernel(x_hbm, i_hbm, o_hbm, idx_v, rows_v, isem, gsem, ssem):
            cid = lax.axis_index("core")
            sid = lax.axis_index("subcore")
            wid = sid * 2 + cid
            row0 = wid * ROWS_PER_W

            pltpu.async_copy(
                i_hbm.at[pl.ds(row0, ROWS_PER_W)], idx_v, isem
            ).wait()

            def start_row(r, b):
                pltpu.async_copy(
                    x_hbm.at[idx_v.at[r, pl.ds(0, W0)]],
                    rows_v.at[b, pl.ds(0, W0)],
                    gsem.at[b],
                )
                pltpu.async_copy(
                    x_hbm.at[idx_v.at[r, pl.ds(W0, W1)]],
                    rows_v.at[b, pl.ds(W0, W1)],
                    gsem.at[b],
                )

            def wait_row(r, b):
                pltpu.make_async_copy(
                    x_hbm.at[idx_v.at[r, pl.ds(0, W0)]],
                    rows_v.at[b, pl.ds(0, W0)],
                    gsem.at[b],
                ).wait()
                pltpu.make_async_copy(
                    x_hbm.at[idx_v.at[r, pl.ds(W0, W1)]],
                    rows_v.at[b, pl.ds(W0, W1)],
                    gsem.at[b],
                ).wait()

            for b in range(NBUF):
                start_row(b, b)

            @pl.loop(0, ROWS_PER_W, step=NBUF)
            def _(r0):
                for b in range(NBUF):
                    r = r0 + b
                    wait_row(r, b)
                    out_slice = o_hbm.at[row0 + r]
                    src = rows_v.at[b, :, pl.ds(0, EMB_DIM)]
                    pltpu.async_copy(src, out_slice, ssem.at[b])
                    pltpu.make_async_copy(src, out_slice, ssem.at[b]).wait()

                    @pl.when(r + NBUF < ROWS_PER_W)
                    def _():
                        start_row(r + NBUF, b)

        return kernel(table, idx)

    return gather


_gather = _gather_fn()


def kernel(inp, emb_weight):
    idx = jnp.clip(inp, 0, VOCAB - 1)
    tpad = jnp.pad(emb_weight, ((0, 0), (0, EMB_DIM)))
    return _gather(tpad, idx)


# speedup vs baseline: 1.6733x; 1.4103x over previous
import jax
import jax.numpy as jnp
from jax import lax
from jax.experimental import pallas as pl
from jax.experimental.pallas import tpu as pltpu
from jax.experimental.pallas import tpu_sc as plsc
from jax.experimental.layout import Layout, with_layout_constraint

VOCAB = 1000000
EMB_DIM = 64
BATCH = 1024
SEQ = 200

NW = 32
ROWS_PER_W = BATCH // NW
W0 = 128
W1 = SEQ - W0
NBUF = 4


def _gather_fn():
    mesh = plsc.VectorSubcoreMesh(
        core_axis_name="core", subcore_axis_name="subcore"
    )

    @jax.jit
    def gather(table, idx):
        @pl.kernel(
            out_type=jax.ShapeDtypeStruct((BATCH, SEQ, EMB_DIM), table.dtype),
            mesh=mesh,
            scratch_types=[
                pltpu.VMEM((ROWS_PER_W, SEQ), jnp.int32),
                pltpu.VMEM((NBUF, SEQ, EMB_DIM), jnp.float32),
                pltpu.SemaphoreType.DMA,
                pltpu.SemaphoreType.DMA((NBUF,)),
                pltpu.SemaphoreType.DMA((NBUF,)),
            ],
            compiler_params=pltpu.CompilerParams(use_tc_tiling_on_sc=False),
        )
        def kernel(x_hbm, i_hbm, o_hbm, idx_v, rows_v, isem, gsem, ssem):
            cid = lax.axis_index("core")
            sid = lax.axis_index("subcore")
            wid = sid * 2 + cid
            row0 = wid * ROWS_PER_W

            pltpu.async_copy(
                i_hbm.at[pl.ds(row0, ROWS_PER_W)], idx_v, isem
            ).wait()

            def start_row(r, b):
                pltpu.async_copy(
                    x_hbm.at[idx_v.at[r, pl.ds(0, W0)]],
                    rows_v.at[b, pl.ds(0, W0)],
                    gsem.at[b],
                )
                pltpu.async_copy(
                    x_hbm.at[idx_v.at[r, pl.ds(W0, W1)]],
                    rows_v.at[b, pl.ds(W0, W1)],
                    gsem.at[b],
                )

            def wait_row(r, b):
                pltpu.make_async_copy(
                    x_hbm.at[idx_v.at[r, pl.ds(0, W0)]],
                    rows_v.at[b, pl.ds(0, W0)],
                    gsem.at[b],
                ).wait()
                pltpu.make_async_copy(
                    x_hbm.at[idx_v.at[r, pl.ds(W0, W1)]],
                    rows_v.at[b, pl.ds(W0, W1)],
                    gsem.at[b],
                ).wait()

            for b in range(NBUF):
                start_row(b, b)

            @pl.loop(0, ROWS_PER_W, step=NBUF)
            def _(r0):
                for b in range(NBUF):
                    r = r0 + b
                    wait_row(r, b)
                    out_slice = o_hbm.at[row0 + r]
                    pltpu.async_copy(rows_v.at[b], out_slice, ssem.at[b])
                    pltpu.make_async_copy(
                        rows_v.at[b], out_slice, ssem.at[b]
                    ).wait()

                    @pl.when(r + NBUF < ROWS_PER_W)
                    def _():
                        start_row(r + NBUF, b)

        return kernel(table, idx)

    return gather


_gather = _gather_fn()

_ROW_MAJOR_T8 = Layout(major_to_minor=(0, 1), tiling=((8,),))


def kernel(inp, emb_weight):
    table = with_layout_constraint(emb_weight, _ROW_MAJOR_T8)
    idx = jnp.clip(inp, 0, VOCAB - 1)
    return _gather(table, idx)
